# baseline (speedup 1.0000x reference)
import functools

import jax
import jax.numpy as jnp
from jax import lax
from jax.experimental import pallas as pl
from jax.experimental.pallas import tpu as pltpu
from jax.experimental.pallas import tpu_sc as plsc

VOCAB = 5000
VOCAB_PAD = 5008
EMBED = 16
B = 16384
L = 200

NC = 2
NS = 16
NW = NC * NS
RPW = B // NW
G = 16
CH = 128
NCH = RPW // CH
GPC = CH // G


def _tw_body(table_ref, w_ref, b_ref, out_ref):
    out_ref[...] = (
        jnp.dot(table_ref[...], w_ref[...], preferred_element_type=jnp.float32)
        + b_ref[0, 0]
    )


def _compute_tw(emb_table, W, b):
    table_pad = jnp.zeros((VOCAB_PAD, EMBED), jnp.float32).at[:VOCAB].set(emb_table)
    tw = pl.pallas_call(
        _tw_body,
        out_shape=jax.ShapeDtypeStruct((VOCAB_PAD, 1), jnp.float32),
    )(table_pad, W, b.reshape(1, 1))
    return tw.reshape(VOCAB_PAD)


def _sc_body(tw_hbm, idx_hbm, out_hbm, tw_v, idx_v, out_v, sem0, sem1):
    wid = lax.axis_index("c") * NS + lax.axis_index("s")
    base = wid * RPW
    sems = (sem0, sem1)

    pltpu.sync_copy(tw_hbm, tw_v)
    cps = [
        pltpu.async_copy(idx_hbm.at[pl.ds(base, CH), :], idx_v.at[0], sems[0]),
        None,
    ]

    lane = lax.iota(jnp.int32, 16)

    for ch in range(NCH):
        cur = ch & 1
        if ch + 1 < NCH:
            nxt = 1 - cur
            cps[nxt] = pltpu.async_copy(
                idx_hbm.at[pl.ds(base + (ch + 1) * CH, CH), :],
                idx_v.at[nxt],
                sems[nxt],
            )
        cps[cur].wait()
        idx_ch = idx_v.at[cur]

        for g in range(GPC):
            rows = lane + g * G

            def step(l, acc):
                tok = plsc.load_gather(idx_ch, [rows, jnp.full((16,), l, jnp.int32)])
                val = plsc.load_gather(tw_v, [tok])
                return acc + val

            acc = lax.fori_loop(0, L, step, jnp.zeros((16,), jnp.float32), unroll=8)
            m = acc * (1.0 / L)
            out_v[pl.ds((ch * GPC + g) * G, G)] = 1.0 / (1.0 + jnp.exp(-m))

    pltpu.sync_copy(out_v, out_hbm.at[pl.ds(base, RPW)])


@functools.partial(
    pl.kernel,
    mesh=plsc.VectorSubcoreMesh(core_axis_name="c", subcore_axis_name="s"),
    out_type=jax.ShapeDtypeStruct((B,), jnp.float32),
    scratch_types=[
        pltpu.VMEM((VOCAB_PAD,), jnp.float32),
        pltpu.VMEM((2, CH, L), jnp.int32),
        pltpu.VMEM((RPW,), jnp.float32),
        pltpu.SemaphoreType.DMA,
        pltpu.SemaphoreType.DMA,
    ],
    compiler_params=pltpu.CompilerParams(needs_layout_passes=False),
)
def _sc_kernel(tw_hbm, idx_hbm, out_hbm, tw_v, idx_v, out_v, sem0, sem1):
    _sc_body(tw_hbm, idx_hbm, out_hbm, tw_v, idx_v, out_v, sem0, sem1)


def kernel(inputs, emb_table, W, b):
    tw = _compute_tw(emb_table, W, b)
    out = _sc_kernel(tw, inputs.astype(jnp.int32))
    return out.reshape(B, 1)


# speedup vs baseline: 1.4278x; 1.4278x over previous
import functools

import jax
import jax.numpy as jnp
from jax import lax
from jax.experimental import pallas as pl
from jax.experimental.pallas import tpu as pltpu
from jax.experimental.pallas import tpu_sc as plsc

VOCAB = 5000
VOCAB_PAD = 5008
EMBED = 16
B = 16384
L = 200

NC = 2
NS = 16
NW = NC * NS
RPW = B // NW
G = 16
CH = 128
NCH = RPW // CH
GPC = CH // G


def _tw_body(table_ref, w_ref, b_ref, out_ref):
    out_ref[...] = (
        jnp.dot(table_ref[...], w_ref[...], preferred_element_type=jnp.float32)
        + b_ref[0, 0]
    )


def _compute_tw(emb_table, W, b):
    table_pad = jnp.zeros((VOCAB_PAD, EMBED), jnp.float32).at[:VOCAB].set(emb_table)
    tw = pl.pallas_call(
        _tw_body,
        out_shape=jax.ShapeDtypeStruct((VOCAB_PAD, 1), jnp.float32),
    )(table_pad, W, b.reshape(1, 1))
    return tw.reshape(VOCAB_PAD)


def _sc_body(tw_hbm, idx_hbm, out_hbm, tw_v, idx_v, out_v, part_v, sem0, sem1):
    wid = lax.axis_index("c") * NS + lax.axis_index("s")
    base = wid * RPW
    sems = (sem0, sem1)

    pltpu.sync_copy(tw_hbm, tw_v)
    cps = [
        pltpu.async_copy(idx_hbm.at[pl.ds(base, CH), :], idx_v.at[0], sems[0]),
        None,
    ]

    lane = lax.iota(jnp.int32, 16)
    lane16 = lane * G
    tail_keep = lane >= (G - (L - (L // G) * G))
    cols = [c * G for c in range(L // G)] + [L - G]

    for ch in range(NCH):
        cur = ch & 1
        if ch + 1 < NCH:
            nxt = 1 - cur
            cps[nxt] = pltpu.async_copy(
                idx_hbm.at[pl.ds(base + (ch + 1) * CH, CH), :],
                idx_v.at[nxt],
                sems[nxt],
            )
        cps[cur].wait()
        idx_ch = idx_v.at[cur]

        def group(g, carry):
            for r in range(G):
                row = g * G + r
                acc = jnp.zeros((16,), jnp.float32)
                for i, c in enumerate(cols):
                    tok = idx_ch[row, pl.ds(c, G)]
                    val = plsc.load_gather(tw_v, [tok])
                    if i == len(cols) - 1:
                        val = jnp.where(tail_keep, val, 0.0)
                    acc = acc + val
                part_v[pl.ds(r * G, G)] = acc
            s = jnp.zeros((16,), jnp.float32)
            for c in range(G):
                s = s + plsc.load_gather(part_v, [lane16 + c])
            m = s * (1.0 / L)
            y = 1.0 / (1.0 + jnp.exp(-m))
            plsc.store_scatter(out_v, [(ch * GPC + g) * G + lane], y)
            return carry

        lax.fori_loop(0, GPC, group, 0)

    pltpu.sync_copy(out_v, out_hbm.at[pl.ds(base, RPW)])


@functools.partial(
    pl.kernel,
    mesh=plsc.VectorSubcoreMesh(core_axis_name="c", subcore_axis_name="s"),
    out_type=jax.ShapeDtypeStruct((B,), jnp.float32),
    scratch_types=[
        pltpu.VMEM((VOCAB_PAD,), jnp.float32),
        pltpu.VMEM((2, CH, L), jnp.int32),
        pltpu.VMEM((RPW,), jnp.float32),
        pltpu.VMEM((G * G,), jnp.float32),
        pltpu.SemaphoreType.DMA,
        pltpu.SemaphoreType.DMA,
    ],
    compiler_params=pltpu.CompilerParams(needs_layout_passes=False),
)
def _sc_kernel(tw_hbm, idx_hbm, out_hbm, tw_v, idx_v, out_v, part_v, sem0, sem1):
    _sc_body(tw_hbm, idx_hbm, out_hbm, tw_v, idx_v, out_v, part_v, sem0, sem1)


def kernel(inputs, emb_table, W, b):
    tw = _compute_tw(emb_table, W, b)
    out = _sc_kernel(tw, inputs.astype(jnp.int32))
    return out.reshape(B, 1)


# speedup vs baseline: 1.5026x; 1.0524x over previous
import functools

import jax
import jax.numpy as jnp
from jax import lax
from jax.experimental import pallas as pl
from jax.experimental.pallas import tpu as pltpu
from jax.experimental.pallas import tpu_sc as plsc

VOCAB = 5000
VOCAB_PAD = 5008
EMBED = 16
B = 16384
L = 200

NC = 2
NS = 16
NW = NC * NS
RPW = B // NW
G = 16
CH = 128
NCH = RPW // CH
GPC = CH // G


def _tw_body(table_ref, w_ref, b_ref, out_ref):
    out_ref[...] = (
        jnp.sum(table_ref[...] * w_ref[...], axis=1) + b_ref[0, 0]
    )


def _compute_tw(emb_table, W, b):
    table_pad = jnp.zeros((VOCAB_PAD, EMBED), jnp.float32).at[:VOCAB].set(emb_table)
    return pl.pallas_call(
        _tw_body,
        out_shape=jax.ShapeDtypeStruct((VOCAB_PAD,), jnp.float32),
    )(table_pad, W.reshape(1, EMBED), b.reshape(1, 1))


def _sc_body(tw_hbm, idx_hbm, out_hbm, tw_v, idx_v, out_v, part_v, sem0, sem1):
    wid = lax.axis_index("c") * NS + lax.axis_index("s")
    base = wid * RPW
    sems = (sem0, sem1)

    pltpu.sync_copy(tw_hbm, tw_v)
    cps = [
        pltpu.async_copy(idx_hbm.at[pl.ds(base, CH), :], idx_v.at[0], sems[0]),
        None,
    ]

    lane = lax.iota(jnp.int32, 16)
    lane16 = lane * G
    tail_keep = lane >= (G - (L - (L // G) * G))
    cols = [c * G for c in range(L // G)] + [L - G]

    for ch in range(NCH):
        cur = ch & 1
        if ch + 1 < NCH:
            nxt = 1 - cur
            cps[nxt] = pltpu.async_copy(
                idx_hbm.at[pl.ds(base + (ch + 1) * CH, CH), :],
                idx_v.at[nxt],
                sems[nxt],
            )
        cps[cur].wait()
        idx_ch = idx_v.at[cur]

        def group(g, carry):
            for r in range(G):
                row = g * G + r
                acc = jnp.zeros((16,), jnp.float32)
                for i, c in enumerate(cols):
                    tok = idx_ch[row, pl.ds(c, G)]
                    val = plsc.load_gather(tw_v, [tok])
                    if i == len(cols) - 1:
                        val = jnp.where(tail_keep, val, 0.0)
                    acc = acc + val
                part_v[pl.ds(r * G, G)] = acc
            s = jnp.zeros((16,), jnp.float32)
            for c in range(G):
                s = s + plsc.load_gather(part_v, [lane16 + c])
            m = s * (1.0 / L)
            y = 1.0 / (1.0 + jnp.exp(-m))
            plsc.store_scatter(out_v, [(ch * GPC + g) * G + lane], y)
            return carry

        lax.fori_loop(0, GPC, group, 0)

    pltpu.sync_copy(out_v, out_hbm.at[pl.ds(base, RPW)])


@functools.partial(
    pl.kernel,
    mesh=plsc.VectorSubcoreMesh(core_axis_name="c", subcore_axis_name="s"),
    out_type=jax.ShapeDtypeStruct((B,), jnp.float32),
    scratch_types=[
        pltpu.VMEM((VOCAB_PAD,), jnp.float32),
        pltpu.VMEM((2, CH, L), jnp.int32),
        pltpu.VMEM((RPW,), jnp.float32),
        pltpu.VMEM((G * G,), jnp.float32),
        pltpu.SemaphoreType.DMA,
        pltpu.SemaphoreType.DMA,
    ],
    compiler_params=pltpu.CompilerParams(needs_layout_passes=False),
)
def _sc_kernel(tw_hbm, idx_hbm, out_hbm, tw_v, idx_v, out_v, part_v, sem0, sem1):
    _sc_body(tw_hbm, idx_hbm, out_hbm, tw_v, idx_v, out_v, part_v, sem0, sem1)


def kernel(inputs, emb_table, W, b):
    tw = _compute_tw(emb_table, W, b)
    out = _sc_kernel(tw, inputs.astype(jnp.int32))
    return out.reshape(B, 1)


# speedup vs baseline: 1.5497x; 1.0313x over previous
import functools

import jax
import jax.numpy as jnp
from jax import lax
from jax.experimental import pallas as pl
from jax.experimental.pallas import tpu as pltpu
from jax.experimental.pallas import tpu_sc as plsc

VOCAB = 5000
VOCAB_PAD = 5008
EMBED = 16
B = 16384
L = 200

NC = 2
NS = 16
NW = NC * NS
RPW = B // NW
G = 16
CH = 128
NCH = RPW // CH
GPC = CH // G


def _tw_body(table_ref, w_ref, b_ref, out_ref):
    out_ref[...] = (
        jnp.sum(table_ref[...] * w_ref[...], axis=1) + b_ref[0, 0]
    )


def _compute_tw(emb_table, W, b):
    table_pad = jnp.zeros((VOCAB_PAD, EMBED), jnp.float32).at[:VOCAB].set(emb_table)
    return pl.pallas_call(
        _tw_body,
        out_shape=jax.ShapeDtypeStruct((VOCAB_PAD,), jnp.float32),
    )(table_pad, W.reshape(1, EMBED), b.reshape(1, 1))


def _sc_body(tw_hbm, idx_hbm, out_hbm, tw_v, idx_v, out_v, part_v, sem0, sem1):
    wid = lax.axis_index("c") * NS + lax.axis_index("s")
    base = wid * RPW
    sems = (sem0, sem1)

    pltpu.sync_copy(tw_hbm, tw_v)
    cps = [
        pltpu.async_copy(idx_hbm.at[pl.ds(base, CH), :], idx_v.at[0], sems[0]),
        None,
    ]

    lane = lax.iota(jnp.int32, 16)
    lane16 = lane * G
    tail_keep = lane >= (G - (L - (L // G) * G))
    cols = [c * G for c in range(L // G)] + [L - G]

    for ch in range(NCH):
        cur = ch & 1
        if ch + 1 < NCH:
            nxt = 1 - cur
            cps[nxt] = pltpu.async_copy(
                idx_hbm.at[pl.ds(base + (ch + 1) * CH, CH), :],
                idx_v.at[nxt],
                sems[nxt],
            )
        cps[cur].wait()
        idx_ch = idx_v.at[cur]

        def group(g, carry):
            for r in range(G):
                row = g * G + r
                acc0 = jnp.zeros((16,), jnp.float32)
                acc1 = jnp.zeros((16,), jnp.float32)
                for i, c in enumerate(cols):
                    tok = idx_ch[row, pl.ds(c, G)]
                    val = plsc.load_gather(tw_v, [tok])
                    if i == len(cols) - 1:
                        val = jnp.where(tail_keep, val, 0.0)
                    if i % 2 == 0:
                        acc0 = acc0 + val
                    else:
                        acc1 = acc1 + val
                part_v[pl.ds(r * G, G)] = acc0 + acc1
            s = jnp.zeros((16,), jnp.float32)
            for c in range(G):
                s = s + plsc.load_gather(part_v, [lane16 + c])
            m = s * (1.0 / L)
            y = 1.0 / (1.0 + jnp.exp(-m))
            plsc.store_scatter(out_v, [(ch * GPC + g) * G + lane], y)
            return carry

        lax.fori_loop(0, GPC, group, 0)

    pltpu.sync_copy(out_v, out_hbm.at[pl.ds(base, RPW)])


@functools.partial(
    pl.kernel,
    mesh=plsc.VectorSubcoreMesh(core_axis_name="c", subcore_axis_name="s"),
    out_type=jax.ShapeDtypeStruct((B,), jnp.float32),
    scratch_types=[
        pltpu.VMEM((VOCAB_PAD,), jnp.float32),
        pltpu.VMEM((2, CH, L), jnp.int32),
        pltpu.VMEM((RPW,), jnp.float32),
        pltpu.VMEM((G * G,), jnp.float32),
        pltpu.SemaphoreType.DMA,
        pltpu.SemaphoreType.DMA,
    ],
    compiler_params=pltpu.CompilerParams(needs_layout_passes=False),
)
def _sc_kernel(tw_hbm, idx_hbm, out_hbm, tw_v, idx_v, out_v, part_v, sem0, sem1):
    _sc_body(tw_hbm, idx_hbm, out_hbm, tw_v, idx_v, out_v, part_v, sem0, sem1)


def kernel(inputs, emb_table, W, b):
    tw = _compute_tw(emb_table, W, b)
    out = _sc_kernel(tw, inputs.astype(jnp.int32))
    return out.reshape(B, 1)
